# baseline (speedup 1.0000x reference)
import jax
import jax.numpy as jnp
from jax import lax
from jax.experimental import pallas as pl
from jax.experimental.pallas import tpu as pltpu

_LANES = 256
_OUT_LANES = _LANES // 2
_BLOCK_ROWS = 2048


def _round_up(x, m):
    return ((x + m - 1) // m) * m


def _matmul_kernel(x_ref, a_ref, b_ref, out_ref):
    y = jnp.dot(x_ref[...], a_ref[...], preferred_element_type=jnp.float32)
    out_ref[...] = y + b_ref[0]


@jax.jit
def _forward(flat, weight, bias):
    n = flat.shape[0]
    rows = n // _LANES
    v = flat.reshape(rows, _LANES)

    block_rows = min(_BLOCK_ROWS, rows)
    rows_p = _round_up(rows, block_rows)
    if rows_p != rows:
        v = jnp.pad(v, ((0, rows_p - rows), (0, 0)))

    w0 = weight[0, 0]
    w1 = weight[0, 1]
    j = jnp.arange(_OUT_LANES)
    a = jnp.zeros((_LANES, _OUT_LANES), jnp.float32)
    a = a.at[2 * j, j].set(w0).at[2 * j + 1, j].set(w1)

    out = pl.pallas_call(
        _matmul_kernel,
        grid=(rows_p // block_rows,),
        out_shape=jax.ShapeDtypeStruct((rows_p, _OUT_LANES), jnp.float32),
        in_specs=[
            pl.BlockSpec((block_rows, _LANES), lambda i: (i, 0)),
            pl.BlockSpec((_LANES, _OUT_LANES), lambda i: (0, 0)),
            pl.BlockSpec(memory_space=pltpu.SMEM),
        ],
        out_specs=pl.BlockSpec((block_rows, _OUT_LANES), lambda i: (i, 0)),
        compiler_params=pltpu.CompilerParams(
            dimension_semantics=("parallel",)),
    )(v, a, bias)
    return out.reshape(-1)[: n // 2]


def kernel(inputs, data_samples, weight, bias):
    del data_samples
    b = inputs.shape[0]
    flat = inputs.reshape(-1)
    n = flat.shape[0]
    n_p = _round_up(n, 2 * _LANES)
    if n_p != n:
        flat = jnp.pad(flat, (0, n_p - n))
    y = _forward(flat, weight, bias)[:b]
    return y.reshape(b, 1)


# speedup vs baseline: 78.6778x; 78.6778x over previous
import jax
import jax.numpy as jnp
from jax.experimental import pallas as pl
from jax.experimental.pallas import tpu as pltpu

_LANES = 128
_BLOCK_ROWS = 2048


def _round_up(x, m):
    return ((x + m - 1) // m) * m


def _fma_kernel(x_ref, w_ref, b_ref, out_ref):
    x = x_ref[...]
    out_ref[...] = x[0] * w_ref[0] + x[1] * w_ref[1] + b_ref[0]


@jax.jit
def _forward(xt, weight, bias):
    b_sz = xt.shape[1]
    rows = _round_up(b_sz, _LANES) // _LANES
    block_rows = min(_BLOCK_ROWS, rows)
    rows_p = _round_up(rows, block_rows)
    if rows_p * _LANES != b_sz:
        xt = jnp.pad(xt, ((0, 0), (0, rows_p * _LANES - b_sz)))
    v = xt.reshape(2, rows_p, _LANES)

    out = pl.pallas_call(
        _fma_kernel,
        grid=(rows_p // block_rows,),
        out_shape=jax.ShapeDtypeStruct((rows_p, _LANES), jnp.float32),
        in_specs=[
            pl.BlockSpec((2, block_rows, _LANES), lambda i: (0, i, 0)),
            pl.BlockSpec(memory_space=pltpu.SMEM),
            pl.BlockSpec(memory_space=pltpu.SMEM),
        ],
        out_specs=pl.BlockSpec((block_rows, _LANES), lambda i: (i, 0)),
        compiler_params=pltpu.CompilerParams(
            dimension_semantics=("parallel",)),
    )(v, weight.reshape(-1), bias)
    return out.reshape(-1)[:b_sz]


def kernel(inputs, data_samples, weight, bias):
    del data_samples
    b_sz = inputs.shape[0]
    y = _forward(inputs.T, weight, bias)
    return y.reshape(b_sz, 1)
